# baseline (speedup 1.0000x reference)
import functools

import jax
import jax.numpy as jnp
from jax import lax
from jax.experimental import pallas as pl
from jax.experimental.pallas import tpu as pltpu
from jax.experimental.pallas import tpu_sc as plsc

_N = 10000
_E = 320000
_DH = 128
_DW1 = _DH + 16
_NC = 2
_NS = 16
_NW = _NC * _NS
_EPW = _E // _NW
_RPT = _N // _NS


def _sc_agg_body(nchunk, h_hbm, src_hbm, dst_hbm, zsum_hbm, sum_out, acc,
                 sidx, didx, rows0, rows1, rows2, g0, g1, g2, s0, s1, s2):
    cid = lax.axis_index("c")
    sid = lax.axis_index("s")
    wid = sid * _NC + cid
    r0 = sid * _RPT
    rows = (rows0, rows1, rows2)
    gsem = (g0, g1, g2)
    ssem = (s0, s1, s2)
    pltpu.sync_copy(zsum_hbm.at[pl.ds(r0, _RPT)], acc.at[pl.ds(r0, _RPT)])
    pltpu.sync_copy(src_hbm.at[wid], sidx)
    pltpu.sync_copy(dst_hbm.at[wid], didx)
    pltpu.async_copy(h_hbm.at[sidx.at[0]], rows0, g0)
    pltpu.async_copy(h_hbm.at[sidx.at[1]], rows1, g1)
    plsc.subcore_barrier()

    def wait_scatter(c, buf):
        pltpu.make_async_copy(rows[buf], acc.at[didx.at[c]],
                              ssem[buf]).wait()

    def step(c, buf, first=False, last=False):
        pltpu.make_async_copy(h_hbm.at[sidx.at[c]], rows[buf],
                              gsem[buf]).wait()
        pltpu.make_async_copy(rows[buf], acc.at[didx.at[c]],
                              ssem[buf]).start(add=True)
        pbuf = (buf - 1) % 3
        if not first:
            wait_scatter(c, pbuf)
        if not last:
            pltpu.make_async_copy(h_hbm.at[sidx.at[c + 2]], rows[pbuf],
                                  gsem[pbuf]).start()

    tail = (nchunk - 3) % 3 + 3
    step(0, 0, first=True)
    step(1, 1)
    step(2, 2)

    @pl.loop(3, nchunk - tail, step=3)
    def _(c):
        step(c, 0)
        step(c + 1, 1)
        step(c + 2, 2)

    for c in range(nchunk - tail, nchunk):
        step(c, c % 3, last=(c + 2 >= nchunk))
    wait_scatter(0, (nchunk - 1) % 3)

    plsc.subcore_barrier()
    pltpu.sync_copy(acc.at[pl.ds(r0, _RPT)], sum_out.at[cid, pl.ds(r0, _RPT)])


def _sc_aggregate(h, src, dst, chunk):
    dw = h.shape[1]
    nchunk = _EPW // chunk
    mesh = plsc.VectorSubcoreMesh(core_axis_name="c", subcore_axis_name="s")
    zsum = jnp.zeros((_N, dw), jnp.float32)
    k = pl.kernel(
        functools.partial(_sc_agg_body, nchunk),
        out_type=jax.ShapeDtypeStruct((_NC, _N, dw), jnp.float32),
        mesh=mesh,
        scratch_types=[
            pltpu.VMEM_SHARED((_N, dw), jnp.float32),
            pltpu.VMEM((nchunk, chunk), jnp.int32),
            pltpu.VMEM((nchunk, chunk), jnp.int32),
            pltpu.VMEM((chunk, dw), jnp.float32),
            pltpu.VMEM((chunk, dw), jnp.float32),
            pltpu.VMEM((chunk, dw), jnp.float32),
            pltpu.SemaphoreType.DMA,
            pltpu.SemaphoreType.DMA,
            pltpu.SemaphoreType.DMA,
            pltpu.SemaphoreType.DMA,
            pltpu.SemaphoreType.DMA,
            pltpu.SemaphoreType.DMA,
        ],
        compiler_params=pltpu.CompilerParams(use_tc_tiling_on_sc=False),
    )
    src3 = src.reshape(_NW, nchunk, chunk)
    dst3 = dst.reshape(_NW, nchunk, chunk)
    return k(h, src3, dst3, zsum)


def _encode_body(x_ref, wi_ref, bi_ref, q_ref, wq_ref, bq_ref, oc_ref,
                 o_ref):
    q = jnp.dot(q_ref[...], wq_ref[...],
                preferred_element_type=jnp.float32) + bq_ref[...]
    h = jnp.dot(x_ref[...], wi_ref[...],
                preferred_element_type=jnp.float32) + bi_ref[...] + q
    o_ref[...] = jnp.concatenate(
        [h, jnp.broadcast_to(oc_ref[...], (_N, _DW1 - _DH))], axis=-1)


def _norm_residual(h, mean, wl_ref, bl_ref, wr_ref, g_ref, be_ref):
    hout = (jnp.dot(mean, wl_ref[...], preferred_element_type=jnp.float32)
            + bl_ref[...]
            + jnp.dot(h, wr_ref[...], preferred_element_type=jnp.float32))
    m = jnp.mean(hout, axis=-1, keepdims=True)
    d = hout - m
    var = jnp.mean(d * d, axis=-1, keepdims=True)
    y = d * lax.rsqrt(var + 1e-5) * g_ref[...] + be_ref[...]
    return h + y


def _layer0_body(hx_ref, s0_ref, s1_ref, wl_ref, bl_ref, wr_ref, g_ref,
                 be_ref, o_ref, c_ref):
    s = s0_ref[...] + s1_ref[...]
    cnt = jnp.maximum(s[:, _DH:_DH + 1], 1.0)
    mean = s[:, :_DH] / cnt
    h = hx_ref[:, :_DH]
    hn = _norm_residual(h, mean, wl_ref, bl_ref, wr_ref, g_ref, be_ref)
    o_ref[...] = jnp.maximum(hn, 0.0)
    c_ref[...] = cnt


def _final_body(h_ref, s0_ref, s1_ref, c_ref, wl_ref, bl_ref, wr_ref,
                g_ref, be_ref, watt_ref, batt_ref, o_ref, a_ref):
    mean = (s0_ref[...] + s1_ref[...]) / c_ref[...]
    hn = _norm_residual(h_ref[...], mean, wl_ref, bl_ref, wr_ref, g_ref,
                        be_ref)
    o_ref[...] = hn
    logits = jnp.dot(hn, watt_ref[...],
                     preferred_element_type=jnp.float32) + batt_ref[...]
    z = logits - jnp.max(logits, axis=0, keepdims=True)
    e = jnp.exp(z)
    a_ref[...] = e / jnp.sum(e, axis=0, keepdims=True)


def kernel(x, edge_index, edge_attr, query_embedding, W_in, b_in, W_q, b_q,
           Wl0, bl0, Wr0, g0, be0, Wl1, bl1, Wr1, g1, be1, W_att, b_att):
    del edge_attr
    src = edge_index[0]
    dst = edge_index[1]
    f32 = jnp.float32
    sds = jax.ShapeDtypeStruct

    onescol = jnp.zeros((1, _DW1 - _DH), f32).at[0, 0].set(1.0)
    hx = pl.pallas_call(
        _encode_body, out_shape=sds((_N, _DW1), f32))(
            x, W_in, b_in.reshape(1, _DH), query_embedding.reshape(1, -1),
            W_q, b_q.reshape(1, _DH), onescol)

    sums0 = _sc_aggregate(hx, src, dst, chunk=40)

    h1, cnt = pl.pallas_call(
        _layer0_body, out_shape=(sds((_N, _DH), f32), sds((_N, 1), f32)))(
            hx, sums0[0], sums0[1], Wl0, bl0.reshape(1, -1), Wr0,
            g0.reshape(1, -1), be0.reshape(1, -1))

    sums1 = _sc_aggregate(h1, src, dst, chunk=80)

    h2, attn = pl.pallas_call(
        _final_body, out_shape=(sds((_N, _DH), f32), sds((_N, 1), f32)))(
            h1, sums1[0], sums1[1], cnt, Wl1, bl1.reshape(1, -1), Wr1,
            g1.reshape(1, -1), be1.reshape(1, -1), W_att,
            b_att.reshape(1, 1))

    return h2, attn.reshape(-1)
